# baseline (speedup 1.0000x reference)
import functools

import jax
import jax.numpy as jnp
from jax import lax
from jax.experimental import pallas as pl
from jax.experimental.pallas import tpu as pltpu
from jax.experimental.pallas import tpu_sc as plsc


def kernel(user, item, user_table, item_table, W, b):
    B = user.shape[0]
    D = user_table.shape[1]

    info = plsc.get_sparse_core_info()
    NC, NS, L = info.num_cores, info.num_subcores, info.num_lanes
    NW = NC * NS
    b_per_w = B // NW
    CH = 128
    n_chunks = b_per_w // CH

    user_r = user.reshape(NW, n_chunks, CH)
    item_r = item.reshape(NW, n_chunks, CH)
    wb = jnp.concatenate([W.reshape(-1), jnp.broadcast_to(b, (16,))])

    mesh = plsc.VectorSubcoreMesh(core_axis_name="c", subcore_axis_name="s")

    @functools.partial(
        pl.kernel,
        mesh=mesh,
        compiler_params=pltpu.CompilerParams(
            needs_layout_passes=False, use_tc_tiling_on_sc=False),
        out_type=jax.ShapeDtypeStruct((B,), jnp.float32),
        scratch_types=[
            pltpu.VMEM((n_chunks, CH), jnp.int32),
            pltpu.VMEM((n_chunks, CH), jnp.int32),
            pltpu.VMEM((b_per_w, D), jnp.float32),
            pltpu.VMEM((b_per_w, D), jnp.float32),
            pltpu.VMEM((L * b_per_w,), jnp.float32),
            pltpu.VMEM((b_per_w,), jnp.float32),
            pltpu.VMEM((48,), jnp.float32),
            pltpu.SemaphoreType.DMA,
        ],
    )
    def gmf_sc(user_hbm, item_hbm, utab_hbm, itab_hbm, wb_hbm, out_hbm,
               idx_u, idx_i, rows_u, rows_i, tr, acc, wb_v, sem):
        wid = lax.axis_index("s") * NC + lax.axis_index("c")
        base = wid * b_per_w

        pltpu.sync_copy(wb_hbm, wb_v)
        pltpu.sync_copy(user_hbm.at[wid], idx_u)
        pltpu.sync_copy(item_hbm.at[wid], idx_i)

        copies = []
        for j in range(n_chunks):
            copies.append(pltpu.async_copy(
                utab_hbm.at[idx_u.at[j]], rows_u.at[pl.ds(j * CH, CH)], sem))
            copies.append(pltpu.async_copy(
                itab_hbm.at[idx_i.at[j]], rows_i.at[pl.ds(j * CH, CH)], sem))
        for c in copies:
            c.wait()

        w0 = wb_v[pl.ds(0, L)]
        w1 = wb_v[pl.ds(L, L)]
        bias = wb_v[pl.ds(2 * L, L)]
        def body(i, carry):
            u0 = rows_u[i, pl.ds(0, L)]
            u1 = rows_u[i, pl.ds(L, L)]
            v0 = rows_i[i, pl.ds(0, L)]
            v1 = rows_i[i, pl.ds(L, L)]
            t = u0 * v0 * w0 + u1 * v1 * w1
            tr[pl.ds(i * L, L)] = t
            return carry

        lax.fori_loop(0, b_per_w, body, 0)

        col0 = lax.iota(jnp.int32, L) * L
        for j in range(b_per_w // L):
            s = bias
            for k in range(L):
                s = s + plsc.load_gather(tr, [col0 + (j * L * L + k)])
            acc[pl.ds(j * L, L)] = s

        pltpu.sync_copy(acc, out_hbm.at[pl.ds(base, b_per_w)])

    return gmf_sc(user_r, item_r, user_table, item_table, wb)
